# baseline (speedup 1.0000x reference)
import functools

import jax
import jax.numpy as jnp
from jax import lax
from jax.experimental import pallas as pl
from jax.experimental.pallas import tpu as pltpu
from jax.experimental.pallas import tpu_sc as plsc

N = 10000
E = 320000
D_IN = 128
H = 256
HH = H // 2
L_LAYERS = 5
C = 7

NC, NS = 2, 16
W_EDGE = 125
NROW = 3
GRP = 2
NGB = 3
N_WIN = 160
N_GRP = N_WIN // GRP
N_GRP_A = N_GRP + 2
EDGES_PER_TILE = E // NS
PEEL = 4
UNROLL = 6
N_ACC = N
ZROWS = N_ACC // NS
ROW_OUT = 624

_F32 = jnp.float32


def _dot(a, b):
    return lax.dot_general(a, b, (((1,), (0,)), ((), ())),
                           precision=lax.Precision.DEFAULT,
                           preferred_element_type=_F32)


@functools.partial(
    pl.kernel,
    out_type=(jax.ShapeDtypeStruct((N, HH), _F32),
              jax.ShapeDtypeStruct((N, HH), _F32)),
    mesh=plsc.VectorSubcoreMesh(core_axis_name="c", subcore_axis_name="s"),
    scratch_types=[
        [pltpu.VMEM((2 * GRP, W_EDGE), jnp.int32)] * NGB,
        [pltpu.VMEM((W_EDGE, HH), _F32)] * NROW,
        pltpu.VMEM_SHARED((N_ACC, HH), _F32),
        [pltpu.SemaphoreType.DMA] * NGB,
        [pltpu.SemaphoreType.DMA] * NROW,
        [pltpu.SemaphoreType.DMA] * NROW,
    ],
)
def _seg_sum(h0_hbm, h1_hbm, ei_hbm, out0, out1,
             eidx, rows, acc, semi, semg, sems):
    c = lax.axis_index("c")
    s = lax.axis_index("s")
    base = s * N_GRP_A

    def _idx_start(q, b):
        pltpu.async_copy(ei_hbm.at[base + q], eidx[b], semi[b])

    def _idx_wait(q, b):
        pltpu.make_async_copy(ei_hbm.at[base + q], eidx[b], semi[b]).wait()

    def _gather_start(bq, t, b3):
        @pl.when(c == 0)
        def _():
            pltpu.async_copy(h0_hbm.at[eidx[bq].at[2 * t]], rows[b3],
                             semg[b3])

        @pl.when(c == 1)
        def _():
            pltpu.async_copy(h1_hbm.at[eidx[bq].at[2 * t]], rows[b3],
                             semg[b3])

    def _gather_wait(bq, t, b3):
        pltpu.make_async_copy(
            h0_hbm.at[eidx[bq].at[2 * t]], rows[b3], semg[b3]).wait()

    def _scatter_start(b3, bq, t):
        pltpu.async_copy(rows[b3], acc.at[eidx[bq].at[2 * t + 1]], sems[b3],
                         add=True)

    def _scatter_wait(b3, bq, t):
        pltpu.make_async_copy(
            rows[b3], acc.at[eidx[bq].at[2 * t + 1]], sems[b3]).wait()

    def _window(w_first, slot, b3, qb, q1, q2):
        b3n = (b3 + 1) % NROW
        if slot == 1:
            _idx_wait(q1, (qb + 1) % NGB)
        if not w_first:
            _scatter_wait(b3n, (qb + 2) % NGB, slot)
        _gather_start((qb + 1) % NGB if slot == 1 else qb, (slot + 1) % GRP,
                      b3n)
        if slot == 1:
            _idx_start(q2, (qb + 2) % NGB)
        _gather_wait(qb, slot, b3)
        _scatter_start(b3, qb, slot)

    _idx_start(0, 0)
    _idx_start(1, 1)

    zero16 = jnp.zeros((16,), _F32)

    def _zrow(r, carry):
        for k in range(HH // 16):
            rows[0][r, pl.ds(k * 16, 16)] = zero16
        return carry

    lax.fori_loop(0, W_EDGE, _zrow, 0)
    z0 = s * ZROWS
    for j in range(ZROWS // W_EDGE):
        pltpu.sync_copy(rows[0], acc.at[pl.ds(z0 + j * W_EDGE, W_EDGE)])
    plsc.subcore_barrier()

    _idx_wait(0, 0)
    _gather_start(0, 0, 0)

    for w in range(PEEL):
        _window(w < 2, w % GRP, w % NROW, (w // GRP) % NGB,
                w // GRP + 1, w // GRP + 2)

    def _step(it, carry):
        q0 = PEEL // GRP + it * (UNROLL // GRP)
        for j in range(UNROLL):
            q = q0 + j // GRP
            _window(False, j % GRP, (PEEL + j) % NROW,
                    ((PEEL + j) // GRP) % NGB, q + 1, q + 2)
        return carry

    lax.fori_loop(0, (N_WIN - PEEL) // UNROLL, _step, 0)

    _gather_wait(N_GRP % NGB, 0, N_WIN % NROW)
    _idx_wait(N_GRP + 1, (N_GRP + 1) % NGB)
    _scatter_wait((N_WIN - 2) % NROW, (N_GRP - 1) % NGB, 0)
    _scatter_wait((N_WIN - 1) % NROW, (N_GRP - 1) % NGB, 1)
    plsc.subcore_barrier()

    r0 = s * ROW_OUT
    tail = N - NS * ROW_OUT

    @pl.when(c == 0)
    def _():
        pltpu.sync_copy(acc.at[pl.ds(r0, ROW_OUT)],
                        out0.at[pl.ds(r0, ROW_OUT)])

    @pl.when(c == 1)
    def _():
        pltpu.sync_copy(acc.at[pl.ds(r0, ROW_OUT)],
                        out1.at[pl.ds(r0, ROW_OUT)])

    @pl.when((c == 0) & (s == NS - 1))
    def _():
        pltpu.sync_copy(acc.at[pl.ds(NS * ROW_OUT, tail)],
                        out0.at[pl.ds(NS * ROW_OUT, tail)])

    @pl.when((c == 1) & (s == NS - 1))
    def _():
        pltpu.sync_copy(acc.at[pl.ds(NS * ROW_OUT, tail)],
                        out1.at[pl.ds(NS * ROW_OUT, tail)])


R_BLK = 1000
_GRID = (N // R_BLK,)


def _enc_body(x_ref, w_ref, b_ref, o0_ref, o1_ref):
    h = jnp.maximum(_dot(x_ref[...], w_ref[...]) + b_ref[...], 0.0)
    o0_ref[...] = h[:, :HH]
    o1_ref[...] = h[:, HH:]


_encode = pl.pallas_call(
    _enc_body,
    grid=_GRID,
    in_specs=[
        pl.BlockSpec((R_BLK, D_IN), lambda i: (i, 0)),
        pl.BlockSpec((D_IN, H), lambda i: (0, 0)),
        pl.BlockSpec((1, H), lambda i: (0, 0)),
    ],
    out_specs=(pl.BlockSpec((R_BLK, HH), lambda i: (i, 0)),
               pl.BlockSpec((R_BLK, HH), lambda i: (i, 0))),
    out_shape=(jax.ShapeDtypeStruct((N, HH), _F32),
               jax.ShapeDtypeStruct((N, HH), _F32)),
)


def _gin_update(h0_ref, h1_ref, a0_ref, a1_ref, scale_ref,
                wa_ref, ba_ref, wb_ref, bb_ref):
    h = jnp.concatenate([h0_ref[...], h1_ref[...]], axis=1)
    a = jnp.concatenate([a0_ref[...], a1_ref[...]], axis=1)
    z = h * scale_ref[...] + a
    z = jnp.maximum(_dot(z, wa_ref[...]) + ba_ref[...], 0.0)
    return jnp.maximum(_dot(z, wb_ref[...]) + bb_ref[...], 0.0)


def _mlp_body(h0_ref, h1_ref, a0_ref, a1_ref, scale_ref,
              wa_ref, ba_ref, wb_ref, bb_ref, o0_ref, o1_ref):
    o = _gin_update(h0_ref, h1_ref, a0_ref, a1_ref, scale_ref,
                    wa_ref, ba_ref, wb_ref, bb_ref)
    o0_ref[...] = o[:, :HH]
    o1_ref[...] = o[:, HH:]


def _mlp_head_body(h0_ref, h1_ref, a0_ref, a1_ref, scale_ref,
                   wa_ref, ba_ref, wb_ref, bb_ref, wh_ref, bh_ref, o_ref):
    o = _gin_update(h0_ref, h1_ref, a0_ref, a1_ref, scale_ref,
                    wa_ref, ba_ref, wb_ref, bb_ref)
    o_ref[...] = _dot(o, wh_ref[...]) + bh_ref[...]


_mlp_in_specs = [
    pl.BlockSpec((R_BLK, HH), lambda i: (i, 0)),
    pl.BlockSpec((R_BLK, HH), lambda i: (i, 0)),
    pl.BlockSpec((R_BLK, HH), lambda i: (i, 0)),
    pl.BlockSpec((R_BLK, HH), lambda i: (i, 0)),
    pl.BlockSpec((1, H), lambda i: (0, 0)),
    pl.BlockSpec((H, H), lambda i: (0, 0)),
    pl.BlockSpec((1, H), lambda i: (0, 0)),
    pl.BlockSpec((H, H), lambda i: (0, 0)),
    pl.BlockSpec((1, H), lambda i: (0, 0)),
]

_mlp = pl.pallas_call(
    _mlp_body,
    grid=_GRID,
    in_specs=_mlp_in_specs,
    out_specs=(pl.BlockSpec((R_BLK, HH), lambda i: (i, 0)),
               pl.BlockSpec((R_BLK, HH), lambda i: (i, 0))),
    out_shape=(jax.ShapeDtypeStruct((N, HH), _F32),
               jax.ShapeDtypeStruct((N, HH), _F32)),
)

_mlp_head = pl.pallas_call(
    _mlp_head_body,
    grid=_GRID,
    in_specs=_mlp_in_specs + [
        pl.BlockSpec((H, 128), lambda i: (0, 0)),
        pl.BlockSpec((1, 128), lambda i: (0, 0)),
    ],
    out_specs=pl.BlockSpec((R_BLK, 128), lambda i: (i, 0)),
    out_shape=jax.ShapeDtypeStruct((N, 128), _F32),
)


def kernel(x, edge_index, W_enc, b_enc, Wa, ba, Wb, bb, eps, W_head, b_head):
    src = edge_index[0].astype(jnp.int32)
    dst = edge_index[1].astype(jnp.int32)

    srcw = src.reshape(NS, N_GRP, GRP, W_EDGE)
    dstw = dst.reshape(NS, N_GRP, GRP, W_EDGE)
    grp = jnp.stack([srcw, dstw], axis=3).reshape(NS, N_GRP, 2 * GRP, W_EDGE)
    ar = jnp.arange(W_EDGE, dtype=jnp.int32)
    pad_grp = jnp.broadcast_to((ar * 37) % N,
                               (NS, N_GRP_A - N_GRP, 2 * GRP, W_EDGE))
    ei = jnp.concatenate([grp, pad_grp], axis=1).reshape(
        NS * N_GRP_A, 2 * GRP, W_EDGE)

    b_enc_r = b_enc.reshape(1, H)
    scales = (1.0 + eps).reshape(L_LAYERS, 1, 1)
    wh_pad = jnp.zeros((H, 128), _F32).at[:, :C].set(W_head)
    bh_pad = jnp.zeros((1, 128), _F32).at[0, :C].set(b_head)

    h0, h1 = _encode(x, W_enc, b_enc_r)
    for l in range(L_LAYERS):
        a0, a1 = _seg_sum(h0, h1, ei)
        scale = jnp.broadcast_to(scales[l], (1, H))
        if l < L_LAYERS - 1:
            h0, h1 = _mlp(h0, h1, a0, a1, scale,
                          Wa[l], ba[l].reshape(1, H), Wb[l], bb[l].reshape(1, H))
        else:
            logits_pad = _mlp_head(h0, h1, a0, a1, scale,
                                   Wa[l], ba[l].reshape(1, H),
                                   Wb[l], bb[l].reshape(1, H),
                                   wh_pad, bh_pad)
    return logits_pad[:, :C]


# speedup vs baseline: 1.0349x; 1.0349x over previous
import functools

import jax
import jax.numpy as jnp
from jax import lax
from jax.experimental import pallas as pl
from jax.experimental.pallas import tpu as pltpu
from jax.experimental.pallas import tpu_sc as plsc

N = 10000
E = 320000
D_IN = 128
H = 256
HH = H // 2
L_LAYERS = 5
C = 7

NC, NS = 2, 16
W_EDGE = 128
NROW = 3
NIDX = 6
N_WIN = 158
N_WIN_A = N_WIN + 2
EDGES_PER_TILE = E // NS
EPT_A = N_WIN_A * W_EDGE
N_ACC = 10016
ZROWS = N_ACC // NS
ZCHUNK = 128
ROW_OUT = 624

_F32 = jnp.float32


def _dot(a, b):
    return lax.dot_general(a, b, (((1,), (0,)), ((), ())),
                           precision=lax.Precision.DEFAULT,
                           preferred_element_type=_F32)


@functools.partial(
    pl.kernel,
    out_type=(jax.ShapeDtypeStruct((N, HH), _F32),
              jax.ShapeDtypeStruct((N, HH), _F32)),
    mesh=plsc.VectorSubcoreMesh(core_axis_name="c", subcore_axis_name="s"),
    scratch_types=[
        [pltpu.VMEM((2, W_EDGE), jnp.int32)] * NIDX,
        [pltpu.VMEM((W_EDGE, HH), _F32)] * NROW,
        pltpu.VMEM_SHARED((N_ACC, HH), _F32),
        [pltpu.SemaphoreType.DMA] * NIDX,
        [pltpu.SemaphoreType.DMA] * NROW,
        [pltpu.SemaphoreType.DMA] * NROW,
    ],
)
def _seg_sum(h0_hbm, h1_hbm, ei_hbm, out0, out1,
             eidx, rows, acc, semi, semg, sems):
    c = lax.axis_index("c")
    s = lax.axis_index("s")
    base = s * N_WIN_A

    def _idx_start(w, b):
        pltpu.async_copy(ei_hbm.at[base + w], eidx[b], semi[b])

    def _idx_wait(w, b):
        pltpu.make_async_copy(ei_hbm.at[base + w], eidx[b], semi[b]).wait()

    def _gather_start(b6, b3):
        @pl.when(c == 0)
        def _():
            pltpu.async_copy(h0_hbm.at[eidx[b6].at[0]], rows[b3], semg[b3])

        @pl.when(c == 1)
        def _():
            pltpu.async_copy(h1_hbm.at[eidx[b6].at[0]], rows[b3], semg[b3])

    def _gather_wait(b6, b3):
        pltpu.make_async_copy(
            h0_hbm.at[eidx[b6].at[0]], rows[b3], semg[b3]).wait()

    def _scatter_start(b3, b6):
        pltpu.async_copy(rows[b3], acc.at[eidx[b6].at[1]], sems[b3],
                         priority=1, add=True)

    def _scatter_wait(b3, b6):
        pltpu.make_async_copy(
            rows[b3], acc.at[eidx[b6].at[1]], sems[b3]).wait()

    _idx_start(0, 0)
    _idx_start(1, 1)

    zero16 = jnp.zeros((16,), _F32)

    def _zrow(r, carry):
        for k in range(HH // 16):
            rows[0][r, pl.ds(k * 16, 16)] = zero16
        return carry

    lax.fori_loop(0, ZCHUNK, _zrow, 0)
    z0 = s * ZROWS
    for j in range(ZROWS // ZCHUNK):
        pltpu.sync_copy(rows[0], acc.at[pl.ds(z0 + j * ZCHUNK, ZCHUNK)])
    ztail = ZROWS % ZCHUNK
    if ztail:
        pltpu.sync_copy(rows[0].at[pl.ds(0, ztail)],
                        acc.at[pl.ds(z0 + ZROWS - ztail, ztail)])
    plsc.subcore_barrier()

    _idx_wait(0, 0)
    _gather_start(0, 0)
    for w in (0, 1):
        _idx_wait(w + 1, w + 1)
        _gather_start(w + 1, w + 1)
        _idx_start(w + 2, w + 2)
        _gather_wait(w, w)
        _scatter_start(w, w)

    def _step(it, carry):
        w0 = 2 + it * NIDX
        for j in range(NIDX):
            w = w0 + j
            b3w, b6w = (2 + j) % NROW, (2 + j) % NIDX
            b3n, b6n = (3 + j) % NROW, (3 + j) % NIDX
            _idx_wait(w + 1, b6n)
            _scatter_wait(b3n, j)
            _gather_start(b6n, b3n)
            _idx_start(w + 2, (4 + j) % NIDX)
            _gather_wait(b6w, b3w)
            _scatter_start(b3w, b6w)
        return carry

    lax.fori_loop(0, (N_WIN - 2) // NIDX, _step, 0)

    _gather_wait(N_WIN % NIDX, N_WIN % NROW)
    _idx_wait(N_WIN + 1, (N_WIN + 1) % NIDX)
    _scatter_wait((N_WIN - 2) % NROW, (N_WIN - 2) % NIDX)
    _scatter_wait((N_WIN - 1) % NROW, (N_WIN - 1) % NIDX)
    plsc.subcore_barrier()

    r0 = s * ROW_OUT
    tail = N - NS * ROW_OUT

    @pl.when(c == 0)
    def _():
        pltpu.sync_copy(acc.at[pl.ds(r0, ROW_OUT)],
                        out0.at[pl.ds(r0, ROW_OUT)])

    @pl.when(c == 1)
    def _():
        pltpu.sync_copy(acc.at[pl.ds(r0, ROW_OUT)],
                        out1.at[pl.ds(r0, ROW_OUT)])

    @pl.when((c == 0) & (s == NS - 1))
    def _():
        pltpu.sync_copy(acc.at[pl.ds(NS * ROW_OUT, tail)],
                        out0.at[pl.ds(NS * ROW_OUT, tail)])

    @pl.when((c == 1) & (s == NS - 1))
    def _():
        pltpu.sync_copy(acc.at[pl.ds(NS * ROW_OUT, tail)],
                        out1.at[pl.ds(NS * ROW_OUT, tail)])


R_BLK = 1000
_GRID = (N // R_BLK,)


def _enc_body(x_ref, w_ref, b_ref, o0_ref, o1_ref):
    h = jnp.maximum(_dot(x_ref[...], w_ref[...]) + b_ref[...], 0.0)
    o0_ref[...] = h[:, :HH]
    o1_ref[...] = h[:, HH:]


_encode = pl.pallas_call(
    _enc_body,
    grid=_GRID,
    in_specs=[
        pl.BlockSpec((R_BLK, D_IN), lambda i: (i, 0)),
        pl.BlockSpec((D_IN, H), lambda i: (0, 0)),
        pl.BlockSpec((1, H), lambda i: (0, 0)),
    ],
    out_specs=(pl.BlockSpec((R_BLK, HH), lambda i: (i, 0)),
               pl.BlockSpec((R_BLK, HH), lambda i: (i, 0))),
    out_shape=(jax.ShapeDtypeStruct((N, HH), _F32),
               jax.ShapeDtypeStruct((N, HH), _F32)),
)


def _gin_update(h0_ref, h1_ref, a0_ref, a1_ref, scale_ref,
                wa_ref, ba_ref, wb_ref, bb_ref):
    h = jnp.concatenate([h0_ref[...], h1_ref[...]], axis=1)
    a = jnp.concatenate([a0_ref[...], a1_ref[...]], axis=1)
    z = h * scale_ref[...] + a
    z = jnp.maximum(_dot(z, wa_ref[...]) + ba_ref[...], 0.0)
    return jnp.maximum(_dot(z, wb_ref[...]) + bb_ref[...], 0.0)


def _mlp_body(h0_ref, h1_ref, a0_ref, a1_ref, scale_ref,
              wa_ref, ba_ref, wb_ref, bb_ref, o0_ref, o1_ref):
    o = _gin_update(h0_ref, h1_ref, a0_ref, a1_ref, scale_ref,
                    wa_ref, ba_ref, wb_ref, bb_ref)
    o0_ref[...] = o[:, :HH]
    o1_ref[...] = o[:, HH:]


def _mlp_head_body(h0_ref, h1_ref, a0_ref, a1_ref, scale_ref,
                   wa_ref, ba_ref, wb_ref, bb_ref, wh_ref, bh_ref, o_ref):
    o = _gin_update(h0_ref, h1_ref, a0_ref, a1_ref, scale_ref,
                    wa_ref, ba_ref, wb_ref, bb_ref)
    o_ref[...] = _dot(o, wh_ref[...]) + bh_ref[...]


_mlp_in_specs = [
    pl.BlockSpec((R_BLK, HH), lambda i: (i, 0)),
    pl.BlockSpec((R_BLK, HH), lambda i: (i, 0)),
    pl.BlockSpec((R_BLK, HH), lambda i: (i, 0)),
    pl.BlockSpec((R_BLK, HH), lambda i: (i, 0)),
    pl.BlockSpec((1, H), lambda i: (0, 0)),
    pl.BlockSpec((H, H), lambda i: (0, 0)),
    pl.BlockSpec((1, H), lambda i: (0, 0)),
    pl.BlockSpec((H, H), lambda i: (0, 0)),
    pl.BlockSpec((1, H), lambda i: (0, 0)),
]

_mlp = pl.pallas_call(
    _mlp_body,
    grid=_GRID,
    in_specs=_mlp_in_specs,
    out_specs=(pl.BlockSpec((R_BLK, HH), lambda i: (i, 0)),
               pl.BlockSpec((R_BLK, HH), lambda i: (i, 0))),
    out_shape=(jax.ShapeDtypeStruct((N, HH), _F32),
               jax.ShapeDtypeStruct((N, HH), _F32)),
)

_mlp_head = pl.pallas_call(
    _mlp_head_body,
    grid=_GRID,
    in_specs=_mlp_in_specs + [
        pl.BlockSpec((H, 128), lambda i: (0, 0)),
        pl.BlockSpec((1, 128), lambda i: (0, 0)),
    ],
    out_specs=pl.BlockSpec((R_BLK, 128), lambda i: (i, 0)),
    out_shape=jax.ShapeDtypeStruct((N, 128), _F32),
)


def kernel(x, edge_index, W_enc, b_enc, Wa, ba, Wb, bb, eps, W_head, b_head):
    src = edge_index[0].astype(jnp.int32)
    dst = edge_index[1].astype(jnp.int32)

    pad = EPT_A - EDGES_PER_TILE
    ar = jnp.arange(pad, dtype=jnp.int32)
    src_pad = jnp.broadcast_to((ar * 37) % N, (NS, pad))
    dst_pad = jnp.broadcast_to(N + ar % (N_ACC - N), (NS, pad))
    src_p = jnp.concatenate(
        [src.reshape(NS, EDGES_PER_TILE), src_pad], axis=1)
    dst_p = jnp.concatenate(
        [dst.reshape(NS, EDGES_PER_TILE), dst_pad], axis=1)
    ei = jnp.stack([src_p.reshape(NS, N_WIN_A, W_EDGE),
                    dst_p.reshape(NS, N_WIN_A, W_EDGE)],
                   axis=2).reshape(NS * N_WIN_A, 2, W_EDGE)

    b_enc_r = b_enc.reshape(1, H)
    scales = (1.0 + eps).reshape(L_LAYERS, 1, 1)
    wh_pad = jnp.zeros((H, 128), _F32).at[:, :C].set(W_head)
    bh_pad = jnp.zeros((1, 128), _F32).at[0, :C].set(b_head)

    h0, h1 = _encode(x, W_enc, b_enc_r)
    for l in range(L_LAYERS):
        a0, a1 = _seg_sum(h0, h1, ei)
        scale = jnp.broadcast_to(scales[l], (1, H))
        if l < L_LAYERS - 1:
            h0, h1 = _mlp(h0, h1, a0, a1, scale,
                          Wa[l], ba[l].reshape(1, H), Wb[l], bb[l].reshape(1, H))
        else:
            logits_pad = _mlp_head(h0, h1, a0, a1, scale,
                                   Wa[l], ba[l].reshape(1, H),
                                   Wb[l], bb[l].reshape(1, H),
                                   wh_pad, bh_pad)
    return logits_pad[:, :C]


# speedup vs baseline: 1.0473x; 1.0120x over previous
import functools

import jax
import jax.numpy as jnp
from jax import lax
from jax.experimental import pallas as pl
from jax.experimental.pallas import tpu as pltpu
from jax.experimental.pallas import tpu_sc as plsc

N = 10000
E = 320000
D_IN = 128
H = 256
HH = H // 2
L_LAYERS = 5
C = 7

NC, NS = 2, 16
W_EDGE = 128
NROW = 3
NIDX = 6
N_WIN = 158
N_WIN_A = N_WIN + 2
EDGES_PER_TILE = E // NS
EPT_A = N_WIN_A * W_EDGE
N_ACC = 10016
ZROWS = N_ACC // NS
ZCHUNK = 128
ROW_OUT = 624

_F32 = jnp.float32


def _dot(a, b):
    return lax.dot_general(a, b, (((1,), (0,)), ((), ())),
                           precision=lax.Precision.DEFAULT,
                           preferred_element_type=_F32)


@functools.partial(
    pl.kernel,
    out_type=(jax.ShapeDtypeStruct((N, HH), _F32),
              jax.ShapeDtypeStruct((N, HH), _F32)),
    mesh=plsc.VectorSubcoreMesh(core_axis_name="c", subcore_axis_name="s"),
    scratch_types=[
        [pltpu.VMEM((2, W_EDGE), jnp.int32)] * NIDX,
        [pltpu.VMEM((W_EDGE, HH), _F32)] * NROW,
        pltpu.VMEM_SHARED((N_ACC, HH), _F32),
        [pltpu.SemaphoreType.DMA] * NIDX,
        [pltpu.SemaphoreType.DMA] * NROW,
        [pltpu.SemaphoreType.DMA] * NROW,
    ],
)
def _seg_sum(h0_hbm, h1_hbm, ei_hbm, out0, out1,
             eidx, rows, acc, semi, semg, sems):
    c = lax.axis_index("c")
    s = lax.axis_index("s")
    base = s * N_WIN_A

    def _idx_start(w, b):
        pltpu.async_copy(ei_hbm.at[base + w], eidx[b], semi[b])

    def _idx_wait(w, b):
        pltpu.make_async_copy(ei_hbm.at[base + w], eidx[b], semi[b]).wait()

    def _gather_start(b6, b3):
        @pl.when(c == 0)
        def _():
            pltpu.async_copy(h0_hbm.at[eidx[b6].at[0]], rows[b3], semg[b3])

        @pl.when(c == 1)
        def _():
            pltpu.async_copy(h1_hbm.at[eidx[b6].at[0]], rows[b3], semg[b3])

    def _gather_wait(b6, b3):
        pltpu.make_async_copy(
            h0_hbm.at[eidx[b6].at[0]], rows[b3], semg[b3]).wait()

    def _scatter_start(b3, b6):
        pltpu.async_copy(rows[b3], acc.at[eidx[b6].at[1]], sems[b3], add=True)

    def _scatter_wait(b3, b6):
        pltpu.make_async_copy(
            rows[b3], acc.at[eidx[b6].at[1]], sems[b3]).wait()

    _idx_start(0, 0)
    _idx_start(1, 1)

    zero16 = jnp.zeros((16,), _F32)

    def _zrow(r, carry):
        for k in range(HH // 16):
            rows[0][r, pl.ds(k * 16, 16)] = zero16
        return carry

    lax.fori_loop(0, ZCHUNK, _zrow, 0)
    z0 = s * ZROWS
    for j in range(ZROWS // ZCHUNK):
        pltpu.sync_copy(rows[0], acc.at[pl.ds(z0 + j * ZCHUNK, ZCHUNK)])
    ztail = ZROWS % ZCHUNK
    if ztail:
        pltpu.sync_copy(rows[0].at[pl.ds(0, ztail)],
                        acc.at[pl.ds(z0 + ZROWS - ztail, ztail)])
    plsc.subcore_barrier()

    _idx_wait(0, 0)
    _gather_start(0, 0)
    for w in (0, 1):
        _idx_wait(w + 1, w + 1)
        _gather_start(w + 1, w + 1)
        _idx_start(w + 2, w + 2)
        _gather_wait(w, w)
        _scatter_start(w, w)

    def _step(it, carry):
        w0 = 2 + it * NIDX
        for j in range(NIDX):
            w = w0 + j
            b3w, b6w = (2 + j) % NROW, (2 + j) % NIDX
            b3n, b6n = (3 + j) % NROW, (3 + j) % NIDX
            _idx_wait(w + 1, b6n)
            _scatter_wait(b3n, j)
            _gather_start(b6n, b3n)
            _idx_start(w + 2, (4 + j) % NIDX)
            _gather_wait(b6w, b3w)
            _scatter_start(b3w, b6w)
        return carry

    lax.fori_loop(0, (N_WIN - 2) // NIDX, _step, 0)

    _gather_wait(N_WIN % NIDX, N_WIN % NROW)
    _idx_wait(N_WIN + 1, (N_WIN + 1) % NIDX)
    _scatter_wait((N_WIN - 2) % NROW, (N_WIN - 2) % NIDX)
    _scatter_wait((N_WIN - 1) % NROW, (N_WIN - 1) % NIDX)
    plsc.subcore_barrier()

    r0 = s * ROW_OUT
    tail = N - NS * ROW_OUT

    @pl.when(c == 0)
    def _():
        pltpu.sync_copy(acc.at[pl.ds(r0, ROW_OUT)],
                        out0.at[pl.ds(r0, ROW_OUT)])

    @pl.when(c == 1)
    def _():
        pltpu.sync_copy(acc.at[pl.ds(r0, ROW_OUT)],
                        out1.at[pl.ds(r0, ROW_OUT)])

    @pl.when((c == 0) & (s == NS - 1))
    def _():
        pltpu.sync_copy(acc.at[pl.ds(NS * ROW_OUT, tail)],
                        out0.at[pl.ds(NS * ROW_OUT, tail)])

    @pl.when((c == 1) & (s == NS - 1))
    def _():
        pltpu.sync_copy(acc.at[pl.ds(NS * ROW_OUT, tail)],
                        out1.at[pl.ds(NS * ROW_OUT, tail)])


R_BLK = 2000
_GRID = (N // R_BLK,)


def _enc_body(x_ref, w_ref, b_ref, o0_ref, o1_ref):
    h = jnp.maximum(_dot(x_ref[...], w_ref[...]) + b_ref[...], 0.0)
    o0_ref[...] = h[:, :HH]
    o1_ref[...] = h[:, HH:]


_encode = pl.pallas_call(
    _enc_body,
    grid=_GRID,
    in_specs=[
        pl.BlockSpec((R_BLK, D_IN), lambda i: (i, 0)),
        pl.BlockSpec((D_IN, H), lambda i: (0, 0)),
        pl.BlockSpec((1, H), lambda i: (0, 0)),
    ],
    out_specs=(pl.BlockSpec((R_BLK, HH), lambda i: (i, 0)),
               pl.BlockSpec((R_BLK, HH), lambda i: (i, 0))),
    out_shape=(jax.ShapeDtypeStruct((N, HH), _F32),
               jax.ShapeDtypeStruct((N, HH), _F32)),
)


def _gin_update(h0_ref, h1_ref, a0_ref, a1_ref, scale_ref,
                wa_ref, ba_ref, wb_ref, bb_ref):
    h = jnp.concatenate([h0_ref[...], h1_ref[...]], axis=1)
    a = jnp.concatenate([a0_ref[...], a1_ref[...]], axis=1)
    z = h * scale_ref[...] + a
    z = jnp.maximum(_dot(z, wa_ref[...]) + ba_ref[...], 0.0)
    return jnp.maximum(_dot(z, wb_ref[...]) + bb_ref[...], 0.0)


def _mlp_body(h0_ref, h1_ref, a0_ref, a1_ref, scale_ref,
              wa_ref, ba_ref, wb_ref, bb_ref, o0_ref, o1_ref):
    o = _gin_update(h0_ref, h1_ref, a0_ref, a1_ref, scale_ref,
                    wa_ref, ba_ref, wb_ref, bb_ref)
    o0_ref[...] = o[:, :HH]
    o1_ref[...] = o[:, HH:]


def _mlp_head_body(h0_ref, h1_ref, a0_ref, a1_ref, scale_ref,
                   wa_ref, ba_ref, wb_ref, bb_ref, wh_ref, bh_ref, o_ref):
    o = _gin_update(h0_ref, h1_ref, a0_ref, a1_ref, scale_ref,
                    wa_ref, ba_ref, wb_ref, bb_ref)
    o_ref[...] = _dot(o, wh_ref[...]) + bh_ref[...]


_mlp_in_specs = [
    pl.BlockSpec((R_BLK, HH), lambda i: (i, 0)),
    pl.BlockSpec((R_BLK, HH), lambda i: (i, 0)),
    pl.BlockSpec((R_BLK, HH), lambda i: (i, 0)),
    pl.BlockSpec((R_BLK, HH), lambda i: (i, 0)),
    pl.BlockSpec((1, H), lambda i: (0, 0)),
    pl.BlockSpec((H, H), lambda i: (0, 0)),
    pl.BlockSpec((1, H), lambda i: (0, 0)),
    pl.BlockSpec((H, H), lambda i: (0, 0)),
    pl.BlockSpec((1, H), lambda i: (0, 0)),
]

_mlp = pl.pallas_call(
    _mlp_body,
    grid=_GRID,
    in_specs=_mlp_in_specs,
    out_specs=(pl.BlockSpec((R_BLK, HH), lambda i: (i, 0)),
               pl.BlockSpec((R_BLK, HH), lambda i: (i, 0))),
    out_shape=(jax.ShapeDtypeStruct((N, HH), _F32),
               jax.ShapeDtypeStruct((N, HH), _F32)),
)

_mlp_head = pl.pallas_call(
    _mlp_head_body,
    grid=_GRID,
    in_specs=_mlp_in_specs + [
        pl.BlockSpec((H, 128), lambda i: (0, 0)),
        pl.BlockSpec((1, 128), lambda i: (0, 0)),
    ],
    out_specs=pl.BlockSpec((R_BLK, 128), lambda i: (i, 0)),
    out_shape=jax.ShapeDtypeStruct((N, 128), _F32),
)


def kernel(x, edge_index, W_enc, b_enc, Wa, ba, Wb, bb, eps, W_head, b_head):
    src = edge_index[0].astype(jnp.int32)
    dst = edge_index[1].astype(jnp.int32)

    pad = EPT_A - EDGES_PER_TILE
    ar = jnp.arange(pad, dtype=jnp.int32)
    src_pad = jnp.broadcast_to((ar * 37) % N, (NS, pad))
    dst_pad = jnp.broadcast_to(N + ar % (N_ACC - N), (NS, pad))
    src_p = jnp.concatenate(
        [src.reshape(NS, EDGES_PER_TILE), src_pad], axis=1)
    dst_p = jnp.concatenate(
        [dst.reshape(NS, EDGES_PER_TILE), dst_pad], axis=1)
    ei = jnp.stack([src_p.reshape(NS, N_WIN_A, W_EDGE),
                    dst_p.reshape(NS, N_WIN_A, W_EDGE)],
                   axis=2).reshape(NS * N_WIN_A, 2, W_EDGE)

    b_enc_r = b_enc.reshape(1, H)
    scales = (1.0 + eps).reshape(L_LAYERS, 1, 1)
    wh_pad = jnp.zeros((H, 128), _F32).at[:, :C].set(W_head)
    bh_pad = jnp.zeros((1, 128), _F32).at[0, :C].set(b_head)

    h0, h1 = _encode(x, W_enc, b_enc_r)
    for l in range(L_LAYERS):
        a0, a1 = _seg_sum(h0, h1, ei)
        scale = jnp.broadcast_to(scales[l], (1, H))
        if l < L_LAYERS - 1:
            h0, h1 = _mlp(h0, h1, a0, a1, scale,
                          Wa[l], ba[l].reshape(1, H), Wb[l], bb[l].reshape(1, H))
        else:
            logits_pad = _mlp_head(h0, h1, a0, a1, scale,
                                   Wa[l], ba[l].reshape(1, H),
                                   Wb[l], bb[l].reshape(1, H),
                                   wh_pad, bh_pad)
    return logits_pad[:, :C]
